# Initial kernel scaffold; baseline (speedup 1.0000x reference)
import jax
import jax.numpy as jnp
from jax.experimental import pallas as pl


def kernel(x, table, W, b):
    raise NotImplementedError("write your pallas kernel here")


# speedup vs baseline: 3.4820x; 3.4820x over previous
import functools
import math

import jax
import jax.numpy as jnp
from jax import lax
from jax.experimental import pallas as pl
from jax.experimental.pallas import tpu as pltpu
from jax.experimental.pallas import tpu_sc as plsc

_EMBED_DIM = 128
_D_MODEL = 1024
_GATHER_WINDOW = 128
_BN = 1024


def _sc_gather(table, idx_flat):
    n = idx_flat.shape[0]
    d = table.shape[1]
    idx2 = idx_flat.reshape(1, n)
    mesh = plsc.VectorSubcoreMesh(core_axis_name="core",
                                  subcore_axis_name="subcore")

    @functools.partial(
        pl.kernel,
        out_type=jax.ShapeDtypeStruct((n, d), table.dtype),
        mesh=mesh,
    )
    def gather_kernel(table_hbm, i_hbm, o_hbm):
        def body(i_vmem, o_vmem):
            pltpu.sync_copy(table_hbm.at[i_vmem.at[0]], o_vmem)

        pltpu.emit_pipeline(
            body,
            grid=(n // _GATHER_WINDOW,),
            in_specs=[pl.BlockSpec((1, _GATHER_WINDOW),
                                   index_map=lambda i: (0, i))],
            out_specs=[pl.BlockSpec((_GATHER_WINDOW, d),
                                    index_map=lambda i: (i, 0))],
            core_axis_name=("core", "subcore"),
            dimension_semantics=(pltpu.PARALLEL,),
        )(i_hbm, o_hbm)

    return gather_kernel(table, idx2)


def _tc_project(emb, W, bias):
    n, d = emb.shape
    m = W.shape[0]
    scale = math.sqrt(d)

    def mm_kernel(e_ref, w_ref, b_ref, o_ref):
        acc = lax.dot_general(
            e_ref[...], w_ref[...],
            dimension_numbers=(((1,), (1,)), ((), ())),
            preferred_element_type=jnp.float32,
        )
        o_ref[...] = acc * scale + b_ref[...]

    return pl.pallas_call(
        mm_kernel,
        grid=(n // _BN,),
        in_specs=[
            pl.BlockSpec((_BN, d), lambda i: (i, 0)),
            pl.BlockSpec((m, d), lambda i: (0, 0)),
            pl.BlockSpec((1, m), lambda i: (0, 0)),
        ],
        out_specs=pl.BlockSpec((_BN, m), lambda i: (i, 0)),
        out_shape=jax.ShapeDtypeStruct((n, m), jnp.float32),
    )(emb, W, bias)


def kernel(x, table, W, b):
    B, L = x.shape
    idx = x.reshape(-1).astype(jnp.int32)
    emb = _sc_gather(table, idx)
    out = _tc_project(emb, W, b.reshape(1, -1))
    return out.reshape(B, L, _D_MODEL)


# speedup vs baseline: 3.4829x; 1.0003x over previous
import functools
import math

import jax
import jax.numpy as jnp
from jax import lax
from jax.experimental import pallas as pl
from jax.experimental.pallas import tpu as pltpu
from jax.experimental.pallas import tpu_sc as plsc

_EMBED_DIM = 128
_D_MODEL = 1024
_GATHER_WINDOW = 128
_BN = 1024


def _sc_gather(table, idx_flat):
    n = idx_flat.shape[0]
    d = table.shape[1]
    idx2 = idx_flat.reshape(1, n)
    mesh = plsc.VectorSubcoreMesh(core_axis_name="core",
                                  subcore_axis_name="subcore")

    @functools.partial(
        pl.kernel,
        out_type=jax.ShapeDtypeStruct((n, d), table.dtype),
        mesh=mesh,
    )
    def gather_kernel(table_hbm, i_hbm, o_hbm):
        def body(i_vmem, o_vmem):
            pltpu.sync_copy(table_hbm.at[i_vmem.at[0]], o_vmem)

        pltpu.emit_pipeline(
            body,
            grid=(n // _GATHER_WINDOW,),
            in_specs=[pl.BlockSpec((1, _GATHER_WINDOW),
                                   index_map=lambda i: (0, i))],
            out_specs=[pl.BlockSpec((_GATHER_WINDOW, d),
                                    index_map=lambda i: (i, 0))],
            core_axis_name=("core", "subcore"),
            dimension_semantics=(pltpu.PARALLEL,),
        )(i_hbm, o_hbm)

    return gather_kernel(table, idx2)


def _tc_project(emb, W, bias):
    n, d = emb.shape
    m = W.shape[0]
    scale = math.sqrt(d)

    def mm_kernel(e_ref, w_ref, b_ref, o_ref):
        acc = lax.dot_general(
            e_ref[...].astype(jnp.bfloat16), w_ref[...].astype(jnp.bfloat16),
            dimension_numbers=(((1,), (1,)), ((), ())),
            preferred_element_type=jnp.float32,
        )
        o_ref[...] = acc * scale + b_ref[...]

    return pl.pallas_call(
        mm_kernel,
        grid=(n // _BN,),
        in_specs=[
            pl.BlockSpec((_BN, d), lambda i: (i, 0)),
            pl.BlockSpec((m, d), lambda i: (0, 0)),
            pl.BlockSpec((1, m), lambda i: (0, 0)),
        ],
        out_specs=pl.BlockSpec((_BN, m), lambda i: (i, 0)),
        out_shape=jax.ShapeDtypeStruct((n, m), jnp.float32),
    )(emb, W, bias)


def kernel(x, table, W, b):
    B, L = x.shape
    idx = x.reshape(-1).astype(jnp.int32)
    emb = _sc_gather(table, idx)
    out = _tc_project(emb, W, b.reshape(1, -1))
    return out.reshape(B, L, _D_MODEL)


# speedup vs baseline: 3.5951x; 1.0322x over previous
import functools
import math

import jax
import jax.numpy as jnp
from jax import lax
from jax.experimental import pallas as pl
from jax.experimental.pallas import tpu as pltpu
from jax.experimental.pallas import tpu_sc as plsc

_EMBED_DIM = 128
_D_MODEL = 1024
_GATHER_WINDOW = 128
_BN = 1024
_NCHUNKS = 4


def _sc_gather(table, idx_flat):
    n = idx_flat.shape[0]
    d = table.shape[1]
    idx2 = idx_flat.reshape(1, n)
    mesh = plsc.VectorSubcoreMesh(core_axis_name="core",
                                  subcore_axis_name="subcore")

    @functools.partial(
        pl.kernel,
        out_type=jax.ShapeDtypeStruct((n, d), table.dtype),
        mesh=mesh,
    )
    def gather_kernel(table_hbm, i_hbm, o_hbm):
        def body(i_vmem, o_vmem):
            pltpu.sync_copy(table_hbm.at[i_vmem.at[0]], o_vmem)

        pltpu.emit_pipeline(
            body,
            grid=(n // _GATHER_WINDOW,),
            in_specs=[pl.BlockSpec((1, _GATHER_WINDOW),
                                   index_map=lambda i: (0, i))],
            out_specs=[pl.BlockSpec((_GATHER_WINDOW, d),
                                    index_map=lambda i: (i, 0))],
            core_axis_name=("core", "subcore"),
            dimension_semantics=(pltpu.PARALLEL,),
        )(i_hbm, o_hbm)

    return gather_kernel(table, idx2)


def _mm_body(e_ref, w_ref, b_ref, o_ref):
    acc = lax.dot_general(
        e_ref[...].astype(jnp.bfloat16), w_ref[...].astype(jnp.bfloat16),
        dimension_numbers=(((1,), (1,)), ((), ())),
        preferred_element_type=jnp.float32,
    )
    o_ref[...] = acc * math.sqrt(_EMBED_DIM) + b_ref[...]


def _tc_project_first(emb, W, bias, n_total):
    nc, d = emb.shape
    m = W.shape[0]

    return pl.pallas_call(
        _mm_body,
        grid=(nc // _BN,),
        in_specs=[
            pl.BlockSpec((_BN, d), lambda i: (i, 0)),
            pl.BlockSpec((m, d), lambda i: (0, 0)),
            pl.BlockSpec((1, m), lambda i: (0, 0)),
        ],
        out_specs=pl.BlockSpec((_BN, m), lambda i: (i, 0)),
        out_shape=jax.ShapeDtypeStruct((n_total, m), jnp.float32),
    )(emb, W, bias)


def _tc_project_inplace(out_prev, emb, W, bias, chunk):
    nc, d = emb.shape
    n_total, m = out_prev.shape
    base = chunk * (nc // _BN)

    def body(_, e_ref, w_ref, b_ref, o_ref):
        _mm_body(e_ref, w_ref, b_ref, o_ref)

    return pl.pallas_call(
        body,
        grid=(nc // _BN,),
        in_specs=[
            pl.BlockSpec(memory_space=pl.ANY),
            pl.BlockSpec((_BN, d), lambda i: (i, 0)),
            pl.BlockSpec((m, d), lambda i: (0, 0)),
            pl.BlockSpec((1, m), lambda i: (0, 0)),
        ],
        out_specs=pl.BlockSpec((_BN, m), lambda i: (base + i, 0)),
        out_shape=jax.ShapeDtypeStruct((n_total, m), jnp.float32),
        input_output_aliases={0: 0},
    )(out_prev, emb, W, bias)


def kernel(x, table, W, b):
    B, L = x.shape
    n = B * L
    nc = n // _NCHUNKS
    idx = x.reshape(-1).astype(jnp.int32)
    bias = b.reshape(1, -1)

    embs = [_sc_gather(table, lax.slice(idx, (i * nc,), ((i + 1) * nc,)))
            for i in range(_NCHUNKS)]
    out = _tc_project_first(embs[0], W, bias, n)
    for i in range(1, _NCHUNKS):
        out = _tc_project_inplace(out, embs[i], W, bias, i)
    return out.reshape(B, L, _D_MODEL)


# speedup vs baseline: 3.9595x; 1.1014x over previous
import functools
import math

import jax
import jax.numpy as jnp
from jax import lax
from jax.experimental import pallas as pl
from jax.experimental.pallas import tpu as pltpu
from jax.experimental.pallas import tpu_sc as plsc

_EMBED_DIM = 128
_D_MODEL = 1024
_GATHER_WINDOW = 128
_BN = 2048
_NCHUNKS = 4


def _sc_gather(table, idx_flat):
    n = idx_flat.shape[0]
    d = table.shape[1]
    idx2 = idx_flat.reshape(1, n)
    mesh = plsc.VectorSubcoreMesh(core_axis_name="core",
                                  subcore_axis_name="subcore")

    @functools.partial(
        pl.kernel,
        out_type=jax.ShapeDtypeStruct((n, d), table.dtype),
        mesh=mesh,
    )
    def gather_kernel(table_hbm, i_hbm, o_hbm):
        def body(i_vmem, o_vmem):
            pltpu.sync_copy(table_hbm.at[i_vmem.at[0]], o_vmem)

        pltpu.emit_pipeline(
            body,
            grid=(n // _GATHER_WINDOW,),
            in_specs=[pl.BlockSpec((1, _GATHER_WINDOW),
                                   index_map=lambda i: (0, i))],
            out_specs=[pl.BlockSpec((_GATHER_WINDOW, d),
                                    index_map=lambda i: (i, 0))],
            core_axis_name=("core", "subcore"),
            dimension_semantics=(pltpu.PARALLEL,),
        )(i_hbm, o_hbm)

    return gather_kernel(table, idx2)


def _mm_body(e_ref, w_ref, b_ref, o_ref):
    acc = lax.dot_general(
        e_ref[...].astype(jnp.bfloat16), w_ref[...].astype(jnp.bfloat16),
        dimension_numbers=(((1,), (1,)), ((), ())),
        preferred_element_type=jnp.float32,
    )
    o_ref[...] = acc * math.sqrt(_EMBED_DIM) + b_ref[...]


def _tc_project_first(emb, W, bias, n_total):
    nc, d = emb.shape
    m = W.shape[0]

    return pl.pallas_call(
        _mm_body,
        grid=(nc // _BN,),
        in_specs=[
            pl.BlockSpec((_BN, d), lambda i: (i, 0)),
            pl.BlockSpec((m, d), lambda i: (0, 0)),
            pl.BlockSpec((1, m), lambda i: (0, 0)),
        ],
        out_specs=pl.BlockSpec((_BN, m), lambda i: (i, 0)),
        out_shape=jax.ShapeDtypeStruct((n_total, m), jnp.float32),
    )(emb, W, bias)


def _tc_project_inplace(out_prev, emb, W, bias, chunk):
    nc, d = emb.shape
    n_total, m = out_prev.shape
    base = chunk * (nc // _BN)

    def body(_, e_ref, w_ref, b_ref, o_ref):
        _mm_body(e_ref, w_ref, b_ref, o_ref)

    return pl.pallas_call(
        body,
        grid=(nc // _BN,),
        in_specs=[
            pl.BlockSpec(memory_space=pl.ANY),
            pl.BlockSpec((_BN, d), lambda i: (i, 0)),
            pl.BlockSpec((m, d), lambda i: (0, 0)),
            pl.BlockSpec((1, m), lambda i: (0, 0)),
        ],
        out_specs=pl.BlockSpec((_BN, m), lambda i: (base + i, 0)),
        out_shape=jax.ShapeDtypeStruct((n_total, m), jnp.float32),
        input_output_aliases={0: 0},
    )(out_prev, emb, W, bias)


def kernel(x, table, W, b):
    B, L = x.shape
    n = B * L
    nc = n // _NCHUNKS
    idx = x.reshape(-1).astype(jnp.int32)
    bias = b.reshape(1, -1)

    embs = [_sc_gather(table, lax.slice(idx, (i * nc,), ((i + 1) * nc,)))
            for i in range(_NCHUNKS)]
    out = _tc_project_first(embs[0], W, bias, n)
    for i in range(1, _NCHUNKS):
        out = _tc_project_inplace(out, embs[i], W, bias, i)
    return out.reshape(B, L, _D_MODEL)


# speedup vs baseline: 4.1111x; 1.0383x over previous
import functools
import math

import jax
import jax.numpy as jnp
from jax import lax
from jax.experimental import pallas as pl
from jax.experimental.pallas import tpu as pltpu
from jax.experimental.pallas import tpu_sc as plsc

_EMBED_DIM = 128
_D_MODEL = 1024
_GATHER_WINDOW = 128
_BN = 4096
_NCHUNKS = 4


def _sc_gather(table, idx_flat):
    n = idx_flat.shape[0]
    d = table.shape[1]
    idx2 = idx_flat.reshape(1, n)
    mesh = plsc.VectorSubcoreMesh(core_axis_name="core",
                                  subcore_axis_name="subcore")

    @functools.partial(
        pl.kernel,
        out_type=jax.ShapeDtypeStruct((n, d), table.dtype),
        mesh=mesh,
    )
    def gather_kernel(table_hbm, i_hbm, o_hbm):
        def body(i_vmem, o_vmem):
            pltpu.sync_copy(table_hbm.at[i_vmem.at[0]], o_vmem)

        pltpu.emit_pipeline(
            body,
            grid=(n // _GATHER_WINDOW,),
            in_specs=[pl.BlockSpec((1, _GATHER_WINDOW),
                                   index_map=lambda i: (0, i))],
            out_specs=[pl.BlockSpec((_GATHER_WINDOW, d),
                                    index_map=lambda i: (i, 0))],
            core_axis_name=("core", "subcore"),
            dimension_semantics=(pltpu.PARALLEL,),
        )(i_hbm, o_hbm)

    return gather_kernel(table, idx2)


def _mm_body(e_ref, w_ref, b_ref, o_ref):
    acc = lax.dot_general(
        e_ref[...].astype(jnp.bfloat16), w_ref[...].astype(jnp.bfloat16),
        dimension_numbers=(((1,), (1,)), ((), ())),
        preferred_element_type=jnp.float32,
    )
    o_ref[...] = acc * math.sqrt(_EMBED_DIM) + b_ref[...]


def _tc_project_first(emb, W, bias, n_total):
    nc, d = emb.shape
    m = W.shape[0]

    return pl.pallas_call(
        _mm_body,
        grid=(nc // _BN,),
        in_specs=[
            pl.BlockSpec((_BN, d), lambda i: (i, 0)),
            pl.BlockSpec((m, d), lambda i: (0, 0)),
            pl.BlockSpec((1, m), lambda i: (0, 0)),
        ],
        out_specs=pl.BlockSpec((_BN, m), lambda i: (i, 0)),
        out_shape=jax.ShapeDtypeStruct((n_total, m), jnp.float32),
    )(emb, W, bias)


def _tc_project_inplace(out_prev, emb, W, bias, chunk):
    nc, d = emb.shape
    n_total, m = out_prev.shape
    base = chunk * (nc // _BN)

    def body(_, e_ref, w_ref, b_ref, o_ref):
        _mm_body(e_ref, w_ref, b_ref, o_ref)

    return pl.pallas_call(
        body,
        grid=(nc // _BN,),
        in_specs=[
            pl.BlockSpec(memory_space=pl.ANY),
            pl.BlockSpec((_BN, d), lambda i: (i, 0)),
            pl.BlockSpec((m, d), lambda i: (0, 0)),
            pl.BlockSpec((1, m), lambda i: (0, 0)),
        ],
        out_specs=pl.BlockSpec((_BN, m), lambda i: (base + i, 0)),
        out_shape=jax.ShapeDtypeStruct((n_total, m), jnp.float32),
        input_output_aliases={0: 0},
    )(out_prev, emb, W, bias)


def kernel(x, table, W, b):
    B, L = x.shape
    n = B * L
    nc = n // _NCHUNKS
    idx = x.reshape(-1).astype(jnp.int32)
    bias = b.reshape(1, -1)

    embs = [_sc_gather(table, lax.slice(idx, (i * nc,), ((i + 1) * nc,)))
            for i in range(_NCHUNKS)]
    out = _tc_project_first(embs[0], W, bias, n)
    for i in range(1, _NCHUNKS):
        out = _tc_project_inplace(out, embs[i], W, bias, i)
    return out.reshape(B, L, _D_MODEL)
